# baseline (speedup 1.0000x reference)
import jax
import jax.numpy as jnp
from jax.experimental import pallas as pl
from jax.experimental.pallas import tpu as pltpu

f32 = jnp.float32
bf16 = jnp.bfloat16


def _dot(a, b):
    return jax.lax.dot_general(a, b, (((1,), (0,)), ((), ())),
                               preferred_element_type=f32)


def kernel(x, adj, adj_high, W0, b0, W1, b1, W2, b2, W_out, b_out):
    N, F = x.shape
    H = W0.shape[1]
    C = W_out.shape[1]
    BM1 = 80
    BM2 = 200
    NBUF1, NBUF1O, NBUF2 = 6, 2, 7
    nb1 = N // BM1
    nb2 = N // BM2

    x_bf = x.astype(bf16)
    W0b = W0.astype(bf16)
    W1x = W1[:F].astype(bf16)
    W1h = W1[F:].astype(bf16)
    Wc_x = jnp.concatenate([W2[:F], W_out[:F]], axis=1).astype(bf16)
    Wc_h0 = jnp.concatenate([W2[F:F + H], W_out[F:F + H]], axis=1).astype(bf16)
    Wc_h1 = jnp.concatenate([W2[F + H:], W_out[F + H:F + 2 * H]], axis=1).astype(bf16)
    Wo2 = W_out[F + 2 * H:].astype(bf16)
    b0r = b0.reshape(1, H).astype(f32)
    b1r = b1.reshape(1, H).astype(f32)
    b2r = b2.reshape(1, H).astype(f32)
    boutr = b_out.reshape(1, C).astype(f32)

    def big_spec(bm, nbuf):
        return pl.BlockSpec((bm, N), lambda i: (i, 0),
                            pipeline_mode=pl.Buffered(buffer_count=nbuf))

    def mega_body(adj_hbm, x_v, w0_v, w1x_v, w1h_v, wcx_v, wch0_v, wch1_v,
                  wo2_v, b0_v, b1_v, b2_v, bout_v,
                  out_v, adjc_hbm,
                  B1s, B2s, B3s, B4s, h0s, parts, cnt):
        B1s[...] = _dot(x_v[...], w0_v[...]).astype(bf16)

        def p1_body(adj_blk, adjc_blk):
            i = cnt[0]
            r = pl.ds(i * BM1, BM1)
            t = adj_blk[...].astype(bf16)
            adjc_blk[...] = t
            h0 = jnp.maximum(_dot(t, B1s[...]) + b0_v[...], 0.0)
            h0b = h0.astype(bf16)
            h0s[r, :] = h0b
            B2s[r, :] = (_dot(x_v[r, :], w1x_v[...])
                         + _dot(h0b, w1h_v[...])).astype(bf16)
            cnt[0] = i + 1

        cnt[0] = 0
        pltpu.emit_pipeline(
            p1_body, grid=(nb1,),
            in_specs=[big_spec(BM1, NBUF1)],
            out_specs=[big_spec(BM1, NBUF1O)],
        )(adj_hbm, adjc_hbm)

        def p234_body(adjc_blk):
            k = cnt[1]
            p = k // nb2
            i = k - p * nb2
            r = pl.ds(i * BM2, BM2)
            t = adjc_blk[...]

            @pl.when(p == 0)
            def _():
                h1 = jnp.maximum(_dot(t, B2s[...]) + b1_v[...], 0.0)
                b3 = (_dot(x_v[r, :], wcx_v[...])
                      + _dot(h0s[r, :], wch0_v[...])
                      + _dot(h1.astype(bf16), wch1_v[...]))
                B3s[r, :] = b3.astype(bf16)

            @pl.when(p == 1)
            def _():
                acc = _dot(t, B3s[...])
                h2 = jnp.maximum(acc[:, :H] + b2_v[...], 0.0)
                parts[r, :] = acc[:, H:] + bout_v[...]
                B4s[r, :] = _dot(h2.astype(bf16), wo2_v[...]).astype(bf16)

            @pl.when(p == 2)
            def _():
                out_v[r, :] = _dot(t, B4s[...]) + parts[r, :]

            cnt[1] = k + 1

        cnt[1] = 0
        pltpu.emit_pipeline(
            p234_body, grid=(3, nb2),
            in_specs=[pl.BlockSpec((BM2, N), lambda p, i: (i, 0),
                                   pipeline_mode=pl.Buffered(buffer_count=NBUF2))],
        )(adjc_hbm)

    vmem = pl.BlockSpec(memory_space=pltpu.MemorySpace.VMEM)
    hbm = pl.BlockSpec(memory_space=pltpu.MemorySpace.HBM)

    out, _ = pl.pallas_call(
        mega_body,
        in_specs=[hbm] + [vmem] * 12,
        out_specs=[vmem, hbm],
        out_shape=[jax.ShapeDtypeStruct((N, C), f32),
                   jax.ShapeDtypeStruct((N, N), bf16)],
        scratch_shapes=[pltpu.VMEM((N, H), bf16),
                        pltpu.VMEM((N, H), bf16),
                        pltpu.VMEM((N, H + C), bf16),
                        pltpu.VMEM((N, C), bf16),
                        pltpu.VMEM((N, H), bf16),
                        pltpu.VMEM((N, C), f32),
                        pltpu.SMEM((4,), jnp.int32)],
    )(adj, x_bf, W0b, W1x, W1h, Wc_x, Wc_h0, Wc_h1, Wo2,
      b0r, b1r, b2r, boutr)

    return out


# speedup vs baseline: 1.0679x; 1.0679x over previous
import jax
import jax.numpy as jnp
from jax.experimental import pallas as pl
from jax.experimental.pallas import tpu as pltpu

f32 = jnp.float32
bf16 = jnp.bfloat16


def _dot(a, b):
    return jax.lax.dot_general(a, b, (((1,), (0,)), ((), ())),
                               preferred_element_type=f32)


def kernel(x, adj, adj_high, W0, b0, W1, b1, W2, b2, W_out, b_out):
    N, F = x.shape
    H = W0.shape[1]
    C = W_out.shape[1]
    BM1 = 80
    BM2 = 400
    NBUF1, NBUF1O, NBUF2 = 7, 2, 3
    nb1 = N // BM1
    nb2 = N // BM2

    x_bf = x.astype(bf16)
    W0b = W0.astype(bf16)
    W1x = W1[:F].astype(bf16)
    W1h = W1[F:].astype(bf16)
    Wc_x = jnp.concatenate([W2[:F], W_out[:F]], axis=1).astype(bf16)
    Wc_h0 = jnp.concatenate([W2[F:F + H], W_out[F:F + H]], axis=1).astype(bf16)
    Wc_h1 = jnp.concatenate([W2[F + H:], W_out[F + H:F + 2 * H]], axis=1).astype(bf16)
    Wo2 = W_out[F + 2 * H:].astype(bf16)
    b0r = b0.reshape(1, H).astype(f32)
    b1r = b1.reshape(1, H).astype(f32)
    b2r = b2.reshape(1, H).astype(f32)
    boutr = b_out.reshape(1, C).astype(f32)

    def big_spec(bm, nbuf):
        return pl.BlockSpec((bm, N), lambda i: (i, 0),
                            pipeline_mode=pl.Buffered(buffer_count=nbuf))

    def mega_body(adj_hbm, x_v, w0_v, w1x_v, w1h_v, wcx_v, wch0_v, wch1_v,
                  wo2_v, b0_v, b1_v, b2_v, bout_v,
                  out_v, adjc_hbm,
                  B1s, B2s, B3s, B4s, h0s, parts, cnt):
        B1s[...] = _dot(x_v[...], w0_v[...]).astype(bf16)

        def p1_body(adj_blk, adjc_blk):
            i = cnt[0]
            r = pl.ds(i * BM1, BM1)
            t = adj_blk[...].astype(bf16)
            adjc_blk[...] = t
            h0 = jnp.maximum(_dot(t, B1s[...]) + b0_v[...], 0.0)
            h0b = h0.astype(bf16)
            h0s[r, :] = h0b
            B2s[r, :] = (_dot(x_v[r, :], w1x_v[...])
                         + _dot(h0b, w1h_v[...])).astype(bf16)
            cnt[0] = i + 1

        cnt[0] = 0
        pltpu.emit_pipeline(
            p1_body, grid=(nb1,),
            in_specs=[big_spec(BM1, NBUF1)],
            out_specs=[big_spec(BM1, NBUF1O)],
        )(adj_hbm, adjc_hbm)

        def p234_body(adjc_blk):
            k = cnt[1]
            p = k // nb2
            i = k - p * nb2
            r = pl.ds(i * BM2, BM2)
            t = adjc_blk[...]

            @pl.when(p == 0)
            def _():
                h1 = jnp.maximum(_dot(t, B2s[...]) + b1_v[...], 0.0)
                b3 = (_dot(x_v[r, :], wcx_v[...])
                      + _dot(h0s[r, :], wch0_v[...])
                      + _dot(h1.astype(bf16), wch1_v[...]))
                B3s[r, :] = b3.astype(bf16)

            @pl.when(p == 1)
            def _():
                acc = _dot(t, B3s[...])
                h2 = jnp.maximum(acc[:, :H] + b2_v[...], 0.0)
                parts[r, :] = acc[:, H:] + bout_v[...]
                B4s[r, :] = _dot(h2.astype(bf16), wo2_v[...]).astype(bf16)

            @pl.when(p == 2)
            def _():
                out_v[r, :] = _dot(t, B4s[...]) + parts[r, :]

            cnt[1] = k + 1

        cnt[1] = 0
        pltpu.emit_pipeline(
            p234_body, grid=(3, nb2),
            in_specs=[pl.BlockSpec((BM2, N), lambda p, i: (i, 0),
                                   pipeline_mode=pl.Buffered(buffer_count=NBUF2))],
        )(adjc_hbm)

    vmem = pl.BlockSpec(memory_space=pltpu.MemorySpace.VMEM)
    hbm = pl.BlockSpec(memory_space=pltpu.MemorySpace.HBM)

    out, _ = pl.pallas_call(
        mega_body,
        in_specs=[hbm] + [vmem] * 12,
        out_specs=[vmem, hbm],
        out_shape=[jax.ShapeDtypeStruct((N, C), f32),
                   jax.ShapeDtypeStruct((N, N), bf16)],
        scratch_shapes=[pltpu.VMEM((N, H), bf16),
                        pltpu.VMEM((N, H), bf16),
                        pltpu.VMEM((N, H + C), bf16),
                        pltpu.VMEM((N, C), bf16),
                        pltpu.VMEM((N, H), bf16),
                        pltpu.VMEM((N, C), f32),
                        pltpu.SMEM((4,), jnp.int32)],
    )(adj, x_bf, W0b, W1x, W1h, Wc_x, Wc_h0, Wc_h1, Wo2,
      b0r, b1r, b2r, boutr)

    return out


# speedup vs baseline: 1.1178x; 1.0467x over previous
import jax
import jax.numpy as jnp
from jax.experimental import pallas as pl
from jax.experimental.pallas import tpu as pltpu

f32 = jnp.float32
bf16 = jnp.bfloat16


def _dot(a, b):
    return jax.lax.dot_general(a, b, (((1,), (0,)), ((), ())),
                               preferred_element_type=f32)


def kernel(x, adj, adj_high, W0, b0, W1, b1, W2, b2, W_out, b_out):
    N, F = x.shape
    H = W0.shape[1]
    C = W_out.shape[1]
    BM1 = 40
    BM2 = 400
    NBUF1, NBUF1O, NBUF2 = 16, 2, 4
    nb1 = N // BM1
    nb2 = N // BM2

    x_bf = x.astype(bf16)
    W0b = W0.astype(bf16)
    W1x = W1[:F].astype(bf16)
    W1h = W1[F:].astype(bf16)
    Wc_x = jnp.concatenate([W2[:F], W_out[:F]], axis=1).astype(bf16)
    Wc_h0 = jnp.concatenate([W2[F:F + H], W_out[F:F + H]], axis=1).astype(bf16)
    Wc_h1 = jnp.concatenate([W2[F + H:], W_out[F + H:F + 2 * H]], axis=1).astype(bf16)
    Wo2 = W_out[F + 2 * H:].astype(bf16)
    b0r = b0.reshape(1, H).astype(f32)
    b1r = b1.reshape(1, H).astype(f32)
    b2r = b2.reshape(1, H).astype(f32)
    boutr = b_out.reshape(1, C).astype(f32)

    def big_spec(bm, nbuf):
        return pl.BlockSpec((bm, N), lambda i: (i, 0),
                            pipeline_mode=pl.Buffered(buffer_count=nbuf))

    def mega_body(adj_hbm, x_v, w0_v, w1x_v, w1h_v, wcx_v, wch0_v, wch1_v,
                  wo2_v, b0_v, b1_v, b2_v, bout_v,
                  out_v, adjc_hbm,
                  B1s, B2s, B3s, B4s, h0s, parts, cnt):
        B1s[...] = _dot(x_v[...], w0_v[...]).astype(bf16)

        def p1_body(adj_blk, adjc_blk):
            i = cnt[0]
            r = pl.ds(i * BM1, BM1)
            t = adj_blk[...].astype(bf16)
            adjc_blk[...] = t
            h0 = jnp.maximum(_dot(t, B1s[...]) + b0_v[...], 0.0)
            h0b = h0.astype(bf16)
            h0s[r, :] = h0b
            B2s[r, :] = (_dot(x_v[r, :], w1x_v[...])
                         + _dot(h0b, w1h_v[...])).astype(bf16)
            cnt[0] = i + 1

        cnt[0] = 0
        pltpu.emit_pipeline(
            p1_body, grid=(nb1,),
            in_specs=[big_spec(BM1, NBUF1)],
            out_specs=[big_spec(BM1, NBUF1O)],
        )(adj_hbm, adjc_hbm)

        def p2_body(adjc_blk):
            i = cnt[1]
            r = pl.ds(i * BM2, BM2)
            h1 = jnp.maximum(_dot(adjc_blk[...], B2s[...]) + b1_v[...], 0.0)
            b3 = (_dot(x_v[r, :], wcx_v[...])
                  + _dot(h0s[r, :], wch0_v[...])
                  + _dot(h1.astype(bf16), wch1_v[...]))
            B3s[r, :] = b3.astype(bf16)
            cnt[1] = i + 1

        cnt[1] = 0
        pltpu.emit_pipeline(
            p2_body, grid=(nb2,),
            in_specs=[big_spec(BM2, NBUF2)],
        )(adjc_hbm)

        def p3_body(adjc_blk):
            i = cnt[2]
            r = pl.ds(i * BM2, BM2)
            acc = _dot(adjc_blk[...], B3s[...])
            h2 = jnp.maximum(acc[:, :H] + b2_v[...], 0.0)
            parts[r, :] = acc[:, H:] + bout_v[...]
            B4s[r, :] = _dot(h2.astype(bf16), wo2_v[...]).astype(bf16)
            cnt[2] = i + 1

        cnt[2] = 0
        pltpu.emit_pipeline(
            p3_body, grid=(nb2,),
            in_specs=[big_spec(BM2, NBUF2)],
        )(adjc_hbm)

        def p4_body(adjc_blk):
            i = cnt[3]
            r = pl.ds(i * BM2, BM2)
            out_v[r, :] = _dot(adjc_blk[...], B4s[...]) + parts[r, :]
            cnt[3] = i + 1

        cnt[3] = 0
        pltpu.emit_pipeline(
            p4_body, grid=(nb2,),
            in_specs=[big_spec(BM2, NBUF2)],
        )(adjc_hbm)

    vmem = pl.BlockSpec(memory_space=pltpu.MemorySpace.VMEM)
    hbm = pl.BlockSpec(memory_space=pltpu.MemorySpace.HBM)

    out, _ = pl.pallas_call(
        mega_body,
        in_specs=[hbm] + [vmem] * 12,
        out_specs=[vmem, hbm],
        out_shape=[jax.ShapeDtypeStruct((N, C), f32),
                   jax.ShapeDtypeStruct((N, N), bf16)],
        scratch_shapes=[pltpu.VMEM((N, H), bf16),
                        pltpu.VMEM((N, H), bf16),
                        pltpu.VMEM((N, H + C), bf16),
                        pltpu.VMEM((N, C), bf16),
                        pltpu.VMEM((N, H), bf16),
                        pltpu.VMEM((N, C), f32),
                        pltpu.SMEM((4,), jnp.int32)],
    )(adj, x_bf, W0b, W1x, W1h, Wc_x, Wc_h0, Wc_h1, Wo2,
      b0r, b1r, b2r, boutr)

    return out


# speedup vs baseline: 1.1285x; 1.0096x over previous
import jax
import jax.numpy as jnp
from jax.experimental import pallas as pl
from jax.experimental.pallas import tpu as pltpu

f32 = jnp.float32
bf16 = jnp.bfloat16


def _dot(a, b):
    return jax.lax.dot_general(a, b, (((1,), (0,)), ((), ())),
                               preferred_element_type=f32)


def kernel(x, adj, adj_high, W0, b0, W1, b1, W2, b2, W_out, b_out):
    N, F = x.shape
    H = W0.shape[1]
    C = W_out.shape[1]
    BM1 = 80
    BM2 = 400
    NBUF1, NBUF1O, NBUF2 = 6, 2, 4
    nb1 = N // BM1
    nb2 = N // BM2

    x_bf = x.astype(bf16)
    W0b = W0.astype(bf16)
    W1x = W1[:F].astype(bf16)
    W1h = W1[F:].astype(bf16)
    Wc_x = jnp.concatenate([W2[:F], W_out[:F]], axis=1).astype(bf16)
    Wc_h0 = jnp.concatenate([W2[F:F + H], W_out[F:F + H]], axis=1).astype(bf16)
    Wc_h1 = jnp.concatenate([W2[F + H:], W_out[F + H:F + 2 * H]], axis=1).astype(bf16)
    Wo2 = W_out[F + 2 * H:].astype(bf16)
    b0r = b0.reshape(1, H).astype(f32)
    b1r = b1.reshape(1, H).astype(f32)
    b2r = b2.reshape(1, H).astype(f32)
    boutr = b_out.reshape(1, C).astype(f32)

    def big_spec(bm, nbuf):
        return pl.BlockSpec((bm, N), lambda i: (i, 0),
                            pipeline_mode=pl.Buffered(buffer_count=nbuf))

    def mega_body(adj_hbm, x_v, w0_v, w1x_v, w1h_v, wcx_v, wch0_v, wch1_v,
                  wo2_v, b0_v, b1_v, b2_v, bout_v,
                  out_v, adjc_hbm,
                  B1s, B2s, B3s, B4s, h0s, h1s, parts, cnt):
        for c in range(4):
            rc = pl.ds(c * (N // 4), N // 4)
            B1s[rc, :] = _dot(x_v[rc, :], w0_v[...]).astype(bf16)

        def p1_body(adj_blk, adjc_blk):
            i = cnt[0]
            r = pl.ds(i * BM1, BM1)
            t = adj_blk[...].astype(bf16)
            adjc_blk[...] = t
            h0 = jnp.maximum(_dot(t, B1s[...]) + b0_v[...], 0.0)
            h0s[r, :] = h0.astype(bf16)
            cnt[0] = i + 1

        cnt[0] = 0
        pltpu.emit_pipeline(
            p1_body, grid=(nb1,),
            in_specs=[big_spec(BM1, NBUF1)],
            out_specs=[big_spec(BM1, NBUF1O)],
        )(adj_hbm, adjc_hbm)

        for c in range(4):
            rc = pl.ds(c * (N // 4), N // 4)
            B2s[rc, :] = (_dot(x_v[rc, :], w1x_v[...])
                          + _dot(h0s[rc, :], w1h_v[...])).astype(bf16)

        def p2_body(adjc_blk):
            i = cnt[1]
            r = pl.ds(i * BM2, BM2)
            h1 = jnp.maximum(_dot(adjc_blk[...], B2s[...]) + b1_v[...], 0.0)
            h1s[r, :] = h1.astype(bf16)
            cnt[1] = i + 1

        cnt[1] = 0
        pltpu.emit_pipeline(
            p2_body, grid=(nb2,),
            in_specs=[big_spec(BM2, NBUF2)],
        )(adjc_hbm)

        for c in range(4):
            rc = pl.ds(c * (N // 4), N // 4)
            B3s[rc, :] = (_dot(x_v[rc, :], wcx_v[...])
                          + _dot(h0s[rc, :], wch0_v[...])
                          + _dot(h1s[rc, :], wch1_v[...])).astype(bf16)

        def p3_body(adjc_blk):
            i = cnt[2]
            r = pl.ds(i * BM2, BM2)
            acc = _dot(adjc_blk[...], B3s[...])
            h2 = jnp.maximum(acc[:, :H] + b2_v[...], 0.0)
            parts[r, :] = (acc[:, H:] + bout_v[...]).astype(bf16)
            B4s[r, :] = _dot(h2.astype(bf16), wo2_v[...]).astype(bf16)
            cnt[2] = i + 1

        cnt[2] = 0
        pltpu.emit_pipeline(
            p3_body, grid=(nb2,),
            in_specs=[big_spec(BM2, NBUF2)],
        )(adjc_hbm)

        def p4_body(adjc_blk):
            i = cnt[3]
            r = pl.ds(i * BM2, BM2)
            out_v[r, :] = _dot(adjc_blk[...], B4s[...]) + parts[r, :].astype(f32)
            cnt[3] = i + 1

        cnt[3] = 0
        pltpu.emit_pipeline(
            p4_body, grid=(nb2,),
            in_specs=[big_spec(BM2, NBUF2)],
        )(adjc_hbm)

    vmem = pl.BlockSpec(memory_space=pltpu.MemorySpace.VMEM)
    hbm = pl.BlockSpec(memory_space=pltpu.MemorySpace.HBM)

    out, _ = pl.pallas_call(
        mega_body,
        in_specs=[hbm] + [vmem] * 12,
        out_specs=[vmem, hbm],
        out_shape=[jax.ShapeDtypeStruct((N, C), f32),
                   jax.ShapeDtypeStruct((N, N), bf16)],
        scratch_shapes=[pltpu.VMEM((N, H), bf16),
                        pltpu.VMEM((N, H), bf16),
                        pltpu.VMEM((N, H + C), bf16),
                        pltpu.VMEM((N, C), bf16),
                        pltpu.VMEM((N, H), bf16),
                        pltpu.VMEM((N, H), bf16),
                        pltpu.VMEM((N, C), bf16),
                        pltpu.SMEM((4,), jnp.int32)],
    )(adj, x_bf, W0b, W1x, W1h, Wc_x, Wc_h0, Wc_h1, Wo2,
      b0r, b1r, b2r, boutr)

    return out


# speedup vs baseline: 1.1293x; 1.0007x over previous
import jax
import jax.numpy as jnp
from jax.experimental import pallas as pl
from jax.experimental.pallas import tpu as pltpu

f32 = jnp.float32
bf16 = jnp.bfloat16


def _dot(a, b):
    return jax.lax.dot_general(a, b, (((1,), (0,)), ((), ())),
                               preferred_element_type=f32)


def kernel(x, adj, adj_high, W0, b0, W1, b1, W2, b2, W_out, b_out):
    N, F = x.shape
    H = W0.shape[1]
    C = W_out.shape[1]
    BM1 = 80
    BM2 = 400
    NBUF1, NBUF1O, NBUF2 = 8, 2, 4
    nb1 = N // BM1
    nb2 = N // BM2

    x_bf = x.astype(bf16)
    W0b = W0.astype(bf16)
    W1x = W1[:F].astype(bf16)
    W1h = W1[F:].astype(bf16)
    Wc_x = jnp.concatenate([W2[:F], W_out[:F]], axis=1).astype(bf16)
    Wc_h0 = jnp.concatenate([W2[F:F + H], W_out[F:F + H]], axis=1).astype(bf16)
    Wc_h1 = jnp.concatenate([W2[F + H:], W_out[F + H:F + 2 * H]], axis=1).astype(bf16)
    Wo2 = W_out[F + 2 * H:].astype(bf16)
    b0r = b0.reshape(1, H).astype(f32)
    b1r = b1.reshape(1, H).astype(f32)
    b2r = b2.reshape(1, H).astype(f32)
    boutr = b_out.reshape(1, C).astype(f32)

    def big_spec(bm, nbuf):
        return pl.BlockSpec((bm, N), lambda i: (i, 0),
                            pipeline_mode=pl.Buffered(buffer_count=nbuf))

    def mega_body(adj_hbm, x_v, w0_v, w1x_v, w1h_v, wcx_v, wch0_v, wch1_v,
                  wo2_v, b0_v, b1_v, b2_v, bout_v,
                  out_v, adjc_hbm,
                  B1s, B2s, B3s, B4s, h0s, h1s, parts, cnt):
        for c in range(4):
            rc = pl.ds(c * (N // 4), N // 4)
            B1s[rc, :] = _dot(x_v[rc, :], w0_v[...]).astype(bf16)

        def p1_body(adj_blk, adjc_blk):
            i = cnt[0]
            r = pl.ds(i * BM1, BM1)
            t = adj_blk[...].astype(bf16)
            adjc_blk[...] = t
            h0 = jnp.maximum(_dot(t, B1s[...]) + b0_v[...], 0.0)
            h0s[r, :] = h0.astype(bf16)
            cnt[0] = i + 1

        cnt[0] = 0
        pltpu.emit_pipeline(
            p1_body, grid=(nb1,),
            in_specs=[big_spec(BM1, NBUF1)],
            out_specs=[big_spec(BM1, NBUF1O)],
        )(adj_hbm, adjc_hbm)

        for c in range(4):
            rc = pl.ds(c * (N // 4), N // 4)
            B2s[rc, :] = (_dot(x_v[rc, :], w1x_v[...])
                          + _dot(h0s[rc, :], w1h_v[...])).astype(bf16)

        def p2_body(adjc_blk):
            i = cnt[1]
            r = pl.ds(i * BM2, BM2)
            h1 = jnp.maximum(_dot(adjc_blk[...], B2s[...]) + b1_v[...], 0.0)
            h1s[r, :] = h1.astype(bf16)
            cnt[1] = i + 1

        cnt[1] = 0
        pltpu.emit_pipeline(
            p2_body, grid=(nb2,),
            in_specs=[big_spec(BM2, NBUF2)],
        )(adjc_hbm)

        for c in range(4):
            rc = pl.ds(c * (N // 4), N // 4)
            B3s[rc, :] = (_dot(x_v[rc, :], wcx_v[...])
                          + _dot(h0s[rc, :], wch0_v[...])
                          + _dot(h1s[rc, :], wch1_v[...])).astype(bf16)

        def p3_body(adjc_blk):
            i = cnt[2]
            r = pl.ds(i * BM2, BM2)
            acc = _dot(adjc_blk[...], B3s[...])
            h2 = jnp.maximum(acc[:, :H] + b2_v[...], 0.0)
            parts[r, :] = (acc[:, H:] + bout_v[...]).astype(bf16)
            B4s[r, :] = _dot(h2.astype(bf16), wo2_v[...]).astype(bf16)
            cnt[2] = i + 1

        cnt[2] = 0
        pltpu.emit_pipeline(
            p3_body, grid=(nb2,),
            in_specs=[big_spec(BM2, NBUF2)],
        )(adjc_hbm)

        def p4_body(adjc_blk):
            i = cnt[3]
            r = pl.ds(i * BM2, BM2)
            out_v[r, :] = _dot(adjc_blk[...], B4s[...]) + parts[r, :].astype(f32)
            cnt[3] = i + 1

        cnt[3] = 0
        pltpu.emit_pipeline(
            p4_body, grid=(nb2,),
            in_specs=[big_spec(BM2, NBUF2)],
        )(adjc_hbm)

    vmem = pl.BlockSpec(memory_space=pltpu.MemorySpace.VMEM)
    hbm = pl.BlockSpec(memory_space=pltpu.MemorySpace.HBM)

    out, _ = pl.pallas_call(
        mega_body,
        in_specs=[hbm] + [vmem] * 12,
        out_specs=[vmem, hbm],
        out_shape=[jax.ShapeDtypeStruct((N, C), f32),
                   jax.ShapeDtypeStruct((N, N), bf16)],
        scratch_shapes=[pltpu.VMEM((N, H), bf16),
                        pltpu.VMEM((N, H), bf16),
                        pltpu.VMEM((N, H + C), bf16),
                        pltpu.VMEM((N, C), bf16),
                        pltpu.VMEM((N, H), bf16),
                        pltpu.VMEM((N, H), bf16),
                        pltpu.VMEM((N, C), bf16),
                        pltpu.SMEM((4,), jnp.int32)],
    )(adj, x_bf, W0b, W1x, W1h, Wc_x, Wc_h0, Wc_h1, Wo2,
      b0r, b1r, b2r, boutr)

    return out
